# Initial kernel scaffold; baseline (speedup 1.0000x reference)
import jax
import jax.numpy as jnp
from jax.experimental import pallas as pl


def kernel(x, conv_w, bn_gamma, bn_beta, bn_mean, bn_var, fusion_weights):
    raise NotImplementedError("write your pallas kernel here")


# speedup vs baseline: 1.3096x; 1.3096x over previous
import numpy as np
import jax
import jax.numpy as jnp
from jax.experimental import pallas as pl
from jax.experimental.pallas import tpu as pltpu

_BN_EPS = 1e-5

_B, _C, _H, _W = 16, 256, 64, 64
_T = _H * _W
_TA = _T // 4
_TB = _T - _TA
_CHUNK = 512
_NCHUNK = _TA // _CHUNK
_OUT_C = 512


def _spa_idx() -> np.ndarray:
    idx = np.arange(_T)
    a_idx = idx[::4]
    b_idx = idx[idx % 4 != 0]
    width = int(np.sqrt(_T))
    ac = np.stack([a_idx // width, a_idx % width], -1).astype(np.float32)
    bc = np.stack([b_idx // width, b_idx % width], -1).astype(np.float32)
    dist = np.sqrt(((ac[:, None, :] - bc[None, :, :]) ** 2).sum(-1))
    return np.argmax(1.0 / (dist + 1e-6), axis=-1)


_SPA = _spa_idx()


def _body(xa_ref, xb_ref, xspa_ref, w_ref, g_ref, be_ref, mu_ref, va_ref,
          fw_ref, o_ref):
    xa = xa_ref[0]
    xb = xb_ref[0]
    xspa = xspa_ref[0]

    an = xa / jnp.sqrt(jnp.sum(xa * xa, axis=0, keepdims=True))
    bn = xb / jnp.sqrt(jnp.sum(xb * xb, axis=0, keepdims=True))

    simT = jax.lax.dot_general(bn, an, (((0,), (0,)), ((), ())),
                               preferred_element_type=jnp.float32)
    m = jnp.max(simT, axis=0, keepdims=True)
    onehot = jnp.where(simT == m, 1.0, 0.0)

    fw = jnp.clip(fw_ref[...], 0.0, 6.0)
    fwn = fw / (jnp.sum(fw) + 1e-8)
    csim = 0.5 * fwn[0, 0]
    cspa = 0.5 * fwn[0, 1]

    sel = jax.lax.dot_general(xb, onehot, (((1,), (0,)), ((), ())),
                              preferred_element_type=jnp.float32)
    fused = (csim + cspa) * xa + cspa * xspa + csim * sel

    out = jax.lax.dot_general(w_ref[...], fused, (((1,), (0,)), ((), ())),
                              preferred_element_type=jnp.float32)
    scale = g_ref[...] / jnp.sqrt(va_ref[...] + _BN_EPS)
    bias = be_ref[...] - mu_ref[...] * scale
    y = out * scale + bias
    o_ref[0] = y * jax.nn.sigmoid(y)


def kernel(x, conv_w, bn_gamma, bn_beta, bn_mean, bn_var, fusion_weights):
    B, C, H, W = x.shape
    x4 = x.reshape(B, C, _TA, 4)
    xa = x4[..., 0]
    xb = x4[..., 1:].reshape(B, C, _TB)
    xspa = xb[:, :, _SPA]

    grid = (B, _NCHUNK)
    out = pl.pallas_call(
        _body,
        grid=grid,
        in_specs=[
            pl.BlockSpec((1, C, _CHUNK), lambda b, i: (b, 0, i)),
            pl.BlockSpec((1, C, _TB), lambda b, i: (b, 0, 0)),
            pl.BlockSpec((1, C, _CHUNK), lambda b, i: (b, 0, i)),
            pl.BlockSpec((_OUT_C, C), lambda b, i: (0, 0)),
            pl.BlockSpec((_OUT_C, 1), lambda b, i: (0, 0)),
            pl.BlockSpec((_OUT_C, 1), lambda b, i: (0, 0)),
            pl.BlockSpec((_OUT_C, 1), lambda b, i: (0, 0)),
            pl.BlockSpec((_OUT_C, 1), lambda b, i: (0, 0)),
            pl.BlockSpec((1, 2), lambda b, i: (0, 0)),
        ],
        out_specs=pl.BlockSpec((1, _OUT_C, _CHUNK), lambda b, i: (b, 0, i)),
        out_shape=jax.ShapeDtypeStruct((B, _OUT_C, _TA), jnp.float32),
        compiler_params=pltpu.CompilerParams(
            dimension_semantics=("parallel", "arbitrary"),
            vmem_limit_bytes=100 * 1024 * 1024,
        ),
    )(
        xa, xb, xspa, conv_w,
        bn_gamma.reshape(_OUT_C, 1), bn_beta.reshape(_OUT_C, 1),
        bn_mean.reshape(_OUT_C, 1), bn_var.reshape(_OUT_C, 1),
        fusion_weights.reshape(1, 2),
    )
    return out.reshape(B, _OUT_C, H // 2, W // 2)
